# baseline (speedup 1.0000x reference)
import functools
import jax
import jax.numpy as jnp
from jax.experimental import pallas as pl

_ENTITY_TYPES = ['card1', 'card2', 'card3', 'card4', 'card5', 'card6',
                 'ProductCD', 'P_emaildomain', 'addr1', 'addr2', 'dist1']
_NREL = len(_ENTITY_TYPES)
_HID = 128


def _txh_body(x_ref, wt_ref, b_ref, o_ref):
    o_ref[:] = jax.nn.relu(
        jnp.dot(x_ref[:], wt_ref[:], preferred_element_type=jnp.float32)
        + b_ref[:])


def _txh(tx_x, tx_W, tx_b):
    n, f = tx_x.shape
    br = 2000
    return pl.pallas_call(
        _txh_body,
        grid=(n // br,),
        in_specs=[
            pl.BlockSpec((br, f), lambda i: (i, 0)),
            pl.BlockSpec((f, _HID), lambda i: (0, 0)),
            pl.BlockSpec((1, _HID), lambda i: (0, 0)),
        ],
        out_specs=pl.BlockSpec((br, _HID), lambda i: (i, 0)),
        out_shape=jax.ShapeDtypeStruct((n, _HID), jnp.float32),
    )(tx_x, tx_W.T, tx_b.reshape(1, _HID))


def _gasd_body(e_ref, wt_ref, at2_ref, g_ref, asd_ref):
    g = jnp.dot(e_ref[:], wt_ref[:], preferred_element_type=jnp.float32)
    g_ref[:] = g
    asd_ref[:] = jnp.dot(g, at2_ref[:], preferred_element_type=jnp.float32)


def _gasd(ent_all, gat_W, att_src, att_dst):
    m = ent_all.shape[0]
    at2 = jnp.zeros((_HID, _HID), jnp.float32)
    at2 = at2.at[:, 0].set(att_src).at[:, 1].set(att_dst)
    br = 2000
    return pl.pallas_call(
        _gasd_body,
        grid=(m // br,),
        in_specs=[
            pl.BlockSpec((br, _HID), lambda i: (i, 0)),
            pl.BlockSpec((_HID, _HID), lambda i: (0, 0)),
            pl.BlockSpec((_HID, _HID), lambda i: (0, 0)),
        ],
        out_specs=[
            pl.BlockSpec((br, _HID), lambda i: (i, 0)),
            pl.BlockSpec((br, _HID), lambda i: (i, 0)),
        ],
        out_shape=[
            jax.ShapeDtypeStruct((m, _HID), jnp.float32),
            jax.ShapeDtypeStruct((m, _HID), jnp.float32),
        ],
    )(ent_all, gat_W.T, at2)


def _mlp_body(c_ref, w1_ref, b1_ref, w2_ref, b2_ref, w3_ref, b3_ref, o_ref):
    h1 = jax.nn.relu(
        jnp.dot(c_ref[:], w1_ref[:], preferred_element_type=jnp.float32)
        + b1_ref[:])
    h2 = jax.nn.relu(
        jnp.dot(h1, w2_ref[:], preferred_element_type=jnp.float32)
        + b2_ref[:])
    o_ref[:] = (jnp.dot(h2, w3_ref[:], preferred_element_type=jnp.float32)
                + b3_ref[:])


def _mlp(combined, W1, b1, W2, b2, W3, b3):
    n, cin = combined.shape
    w3p = jnp.zeros((64, _HID), jnp.float32).at[:, 0].set(W3[0])
    b3p = jnp.zeros((1, _HID), jnp.float32).at[0, 0].set(b3[0])
    br = 2000
    out = pl.pallas_call(
        _mlp_body,
        grid=(n // br,),
        in_specs=[
            pl.BlockSpec((br, cin), lambda i: (i, 0)),
            pl.BlockSpec((cin, _HID), lambda i: (0, 0)),
            pl.BlockSpec((1, _HID), lambda i: (0, 0)),
            pl.BlockSpec((_HID, 64), lambda i: (0, 0)),
            pl.BlockSpec((1, 64), lambda i: (0, 0)),
            pl.BlockSpec((64, _HID), lambda i: (0, 0)),
            pl.BlockSpec((1, _HID), lambda i: (0, 0)),
        ],
        out_specs=pl.BlockSpec((br, _HID), lambda i: (i, 0)),
        out_shape=jax.ShapeDtypeStruct((n, _HID), jnp.float32),
    )(combined, W1.T, b1.reshape(1, _HID), W2.T, b2.reshape(1, 64), w3p, b3p)
    return out[:, :1]


def _edge_phase(G, a_s, a_d, src, dst, gat_bias, n):
    mhat = jnp.maximum(a_s.max(), 0.) + jnp.maximum(a_d.max(), 0.)
    mhat = jnp.where(mhat >= 0, mhat, 0.2 * mhat)
    alpha = a_s[src] + a_d[dst]
    alpha = jnp.where(alpha >= 0, alpha, 0.2 * alpha)
    ex = jnp.exp(alpha - mhat)
    denom = jax.ops.segment_sum(ex, dst, num_segments=n)
    coef = ex / (denom[dst] + 1e-16)
    h1 = jax.ops.segment_sum(G[src] * coef[:, None], dst, num_segments=n)
    cnt = jax.ops.segment_sum(jnp.ones_like(ex), dst, num_segments=n)
    aggraw = jax.ops.segment_sum(h1[src], dst, num_segments=n)
    agg = aggraw / jnp.clip(cnt, 1.0, None)[:, None] \
        + gat_bias[None, :] * (cnt > 0)[:, None]
    return agg


def kernel(tx_x, edge_index_card1, emb_card1, edge_index_card2, emb_card2,
           edge_index_card3, emb_card3, edge_index_card4, emb_card4,
           edge_index_card5, emb_card5, edge_index_card6, emb_card6,
           edge_index_ProductCD, emb_ProductCD,
           edge_index_P_emaildomain, emb_P_emaildomain,
           edge_index_addr1, emb_addr1, edge_index_addr2, emb_addr2,
           edge_index_dist1, emb_dist1,
           tx_W, tx_b, gat_W, gat_att_src, gat_att_dst, gat_bias,
           cls_W1, cls_b1, cls_W2, cls_b2, cls_W3, cls_b3):
    inp = dict(locals())
    n = tx_x.shape[0]
    edges = [inp['edge_index_' + t] for t in _ENTITY_TYPES]
    embs = [inp['emb_' + t] for t in _ENTITY_TYPES]

    ent_all = jnp.concatenate(
        [jnp.take(emb, ei[0, :n], axis=0) for ei, emb in zip(edges, embs)],
        axis=0)
    G_all, ASD_all = _gasd(ent_all, gat_W, gat_att_src, gat_att_dst)

    tx_h = _txh(tx_x, tx_W, tx_b)

    msgs = []
    for r in range(_NREL):
        G = G_all[r * n:(r + 1) * n]
        a_s = ASD_all[r * n:(r + 1) * n, 0]
        a_d = ASD_all[r * n:(r + 1) * n, 1]
        src = edges[r][0]
        dst = edges[r][1]
        msgs.append(_edge_phase(G, a_s, a_d, src, dst, gat_bias, n))

    combined = jnp.concatenate([tx_h] + msgs, axis=-1)
    return _mlp(combined, cls_W1, cls_b1, cls_W2, cls_b2, cls_W3, cls_b3)


# speedup vs baseline: 10.4447x; 10.4447x over previous
import functools
import jax
import jax.numpy as jnp
from jax import lax
from jax.experimental import pallas as pl
from jax.experimental.pallas import tpu as pltpu
from jax.experimental.pallas import tpu_sc as plsc

_ENTITY_TYPES = ['card1', 'card2', 'card3', 'card4', 'card5', 'card6',
                 'ProductCD', 'P_emaildomain', 'addr1', 'addr2', 'dist1']
_R = len(_ENTITY_TYPES)
_HID = 128
_N = 10000
_NP = 10240
_E = 160000
_EPP = 10240
_ERODS = 80
_K = 256
_NCH = _EPP // _K


def _txh_body(x_ref, wt_ref, b_ref, o_ref):
    o_ref[:] = jax.nn.relu(
        jnp.dot(x_ref[:], wt_ref[:], preferred_element_type=jnp.float32)
        + b_ref[:])


def _txh(tx_x, tx_W, tx_b):
    n, f = tx_x.shape
    br = 2000
    return pl.pallas_call(
        _txh_body,
        grid=(n // br,),
        in_specs=[
            pl.BlockSpec((br, f), lambda i: (i, 0)),
            pl.BlockSpec((f, _HID), lambda i: (0, 0)),
            pl.BlockSpec((1, _HID), lambda i: (0, 0)),
        ],
        out_specs=pl.BlockSpec((br, _HID), lambda i: (i, 0)),
        out_shape=jax.ShapeDtypeStruct((n, _HID), jnp.float32),
    )(tx_x, tx_W.T, tx_b.reshape(1, _HID))


def _gasd_body(e_ref, wt_ref, at2_ref, g_ref, asd_ref):
    g = jnp.dot(e_ref[:], wt_ref[:], preferred_element_type=jnp.float32)
    g_ref[:] = g
    asd_ref[:] = jnp.dot(g, at2_ref[:], preferred_element_type=jnp.float32)


def _gasd(ent_all, gat_W, att_src, att_dst):
    m = ent_all.shape[0]
    at2 = jnp.zeros((_HID, _HID), jnp.float32)
    at2 = at2.at[:, 0].set(att_src).at[:, 1].set(att_dst)
    br = 2560
    return pl.pallas_call(
        _gasd_body,
        grid=(m // br,),
        in_specs=[
            pl.BlockSpec((br, _HID), lambda i: (i, 0)),
            pl.BlockSpec((_HID, _HID), lambda i: (0, 0)),
            pl.BlockSpec((_HID, _HID), lambda i: (0, 0)),
        ],
        out_specs=[
            pl.BlockSpec((br, _HID), lambda i: (i, 0)),
            pl.BlockSpec((br, _HID), lambda i: (i, 0)),
        ],
        out_shape=[
            jax.ShapeDtypeStruct((m, _HID), jnp.float32),
            jax.ShapeDtypeStruct((m, _HID), jnp.float32),
        ],
    )(ent_all, gat_W.T, at2)


def _mlp_body(c_ref, w1_ref, b1_ref, w2_ref, b2_ref, w3_ref, b3_ref, o_ref):
    h1 = jax.nn.relu(
        jnp.dot(c_ref[:], w1_ref[:], preferred_element_type=jnp.float32)
        + b1_ref[:])
    h2 = jax.nn.relu(
        jnp.dot(h1, w2_ref[:], preferred_element_type=jnp.float32)
        + b2_ref[:])
    o_ref[:] = (jnp.dot(h2, w3_ref[:], preferred_element_type=jnp.float32)
                + b3_ref[:])


def _mlp(combined, W1, b1, W2, b2, W3, b3):
    n, cin = combined.shape
    w3p = jnp.zeros((64, _HID), jnp.float32).at[:, 0].set(W3[0])
    b3p = jnp.zeros((1, _HID), jnp.float32).at[0, 0].set(b3[0])
    br = 2000
    out = pl.pallas_call(
        _mlp_body,
        grid=(n // br,),
        in_specs=[
            pl.BlockSpec((br, cin), lambda i: (i, 0)),
            pl.BlockSpec((cin, _HID), lambda i: (0, 0)),
            pl.BlockSpec((1, _HID), lambda i: (0, 0)),
            pl.BlockSpec((_HID, 64), lambda i: (0, 0)),
            pl.BlockSpec((1, 64), lambda i: (0, 0)),
            pl.BlockSpec((64, _HID), lambda i: (0, 0)),
            pl.BlockSpec((1, _HID), lambda i: (0, 0)),
        ],
        out_specs=pl.BlockSpec((br, _HID), lambda i: (i, 0)),
        out_shape=jax.ShapeDtypeStruct((n, _HID), jnp.float32),
    )(combined, W1.T, b1.reshape(1, _HID), W2.T, b2.reshape(1, 64), w3p, b3p)
    return out[:, :1]


def _ent_gather(src_flat, embs):
    mesh = plsc.VectorSubcoreMesh(core_axis_name="c", subcore_axis_name="s")

    @functools.partial(
        pl.kernel,
        out_type=jax.ShapeDtypeStruct((_R * _NP, 128), jnp.float32),
        mesh=mesh,
        compiler_params=pltpu.CompilerParams(needs_layout_passes=False),
        scratch_types=[
            pltpu.VMEM((64,), jnp.int32),
            pltpu.VMEM((64, 128), jnp.float32),
            pltpu.SemaphoreType.DMA,
        ],
    )
    def k(src_h, *args):
        emb_hs = args[:_R]
        ent_h = args[_R]
        gidx_v, grow_v, sem = args[_R + 1:]
        w = lax.axis_index("c") * 16 + lax.axis_index("s")
        boff = w * 320
        for t in range(_R):
            def _chunk(q, cr, t=t):
                pltpu.sync_copy(
                    src_h.at[pl.ds(t * _E + boff + q * 64, 64)], gidx_v)
                pltpu.async_copy(emb_hs[t].at[gidx_v], grow_v, sem).wait()
                pltpu.sync_copy(
                    grow_v, ent_h.at[pl.ds(t * _NP + boff + q * 64, 64)])
                return cr
            lax.fori_loop(0, 5, _chunk, 0)

    return k(src_flat, *embs)


def _edge_sc(srcp, dstp, as_flat, ad_flat, mh_flat, g2a, g2b, bias,
             zeros1, zeros2):
    mesh = plsc.VectorSubcoreMesh(core_axis_name="c", subcore_axis_name="s")

    @functools.partial(
        pl.kernel,
        out_type=[jax.ShapeDtypeStruct((2 * _R * _NP, 64), jnp.float32),
                  jax.ShapeDtypeStruct((2 * _NP, 64), jnp.float32)],
        mesh=mesh,
        compiler_params=pltpu.CompilerParams(
            needs_layout_passes=False, use_tc_tiling_on_sc=False),
        scratch_types=[
            pltpu.VMEM((_EPP,), jnp.int32),
            pltpu.VMEM((_EPP,), jnp.int32),
            pltpu.VMEM((_NP,), jnp.float32),
            pltpu.VMEM((_NP,), jnp.float32),
            pltpu.VMEM((_EPP,), jnp.float32),
            pltpu.VMEM((_EPP,), jnp.float32),
            pltpu.VMEM((_K,), jnp.float32),
            pltpu.VMEM((_K,), jnp.int32),
            pltpu.VMEM((_K,), jnp.int32),
            pltpu.VMEM((_K,), jnp.int32),
            pltpu.VMEM((_K, 64), jnp.float32),
            pltpu.VMEM((16,), jnp.float32),
            pltpu.VMEM((64,), jnp.float32),
            pltpu.VMEM((640,), jnp.float32),
            pltpu.VMEM_SHARED((_NP,), jnp.float32),
            pltpu.VMEM_SHARED((_NP,), jnp.float32),
            pltpu.VMEM_SHARED((_NP, 64), jnp.float32),
            pltpu.SemaphoreType.DMA,
        ],
    )
    def k(srcp_h, dstp_h, as_h, ad_h, mh_h, g2a_h, g2b_h, bias_h,
          z1_h, z2_h, out_h, h1t_h,
          src_v, dst_v, as_v, ad_v, ex_v, den_e, ones_v, idxg_v, didx_v,
          sidx_v, rows_v, mh_v, bias_v, cnt_r, den_sh, cnt_sh, acc_sh,
          sem):
        c = lax.axis_index("c")
        s = lax.axis_index("s")
        pltpu.sync_copy(bias_h.at[pl.ds(c * 64, 64)], bias_v)

        def _ones(i, carry):
            ones_v[pl.ds(i * 16, 16)] = jnp.zeros((16,), jnp.float32) + 1.0
            return carry
        lax.fori_loop(0, _K // 16, _ones, 0)

        def _rel(r, carry):
            pltpu.sync_copy(z1_h, den_sh.at[pl.ds(s * 640, 640)])
            pltpu.sync_copy(z1_h, cnt_sh.at[pl.ds(s * 640, 640)])
            pltpu.sync_copy(z2_h, acc_sh.at[pl.ds(s * 640, 640)])
            coff = (r * 16 + s) * _EPP
            pltpu.sync_copy(srcp_h.at[pl.ds(coff, _EPP)], src_v)
            pltpu.sync_copy(dstp_h.at[pl.ds(coff, _EPP)], dst_v)
            pltpu.sync_copy(as_h.at[pl.ds(r * _NP, _NP)], as_v)
            pltpu.sync_copy(ad_h.at[pl.ds(r * _NP, _NP)], ad_v)
            pltpu.sync_copy(mh_h.at[pl.ds(r * 16, 16)], mh_v)
            plsc.subcore_barrier()
            mh = mh_v[:]

            def _alpha(i, cr):
                sl = pl.ds(i * 16, 16)
                sv = src_v[sl]
                dv = dst_v[sl]
                al = (plsc.load_gather(as_v, [sv])
                      + plsc.load_gather(ad_v, [dv]))
                al = jnp.where(al >= 0.0, al, al * 0.2)
                ex_v[sl] = jnp.exp(al - mh)
                return cr
            lax.fori_loop(0, _EPP // 16, _alpha, 0)

            def _dsc(b, cr):
                base = b * _K

                def _mkidx(j, c2):
                    sl = pl.ds(j * 16, 16)
                    didx_v[sl] = dst_v[pl.ds(base + j * 16, 16)]
                    return c2
                lax.fori_loop(0, _K // 16, _mkidx, 0)
                pltpu.sync_copy(ex_v.at[pl.ds(base, _K)],
                                den_sh.at[didx_v], add=True)
                pltpu.sync_copy(ones_v, cnt_sh.at[didx_v], add=True)
                return cr
            lax.fori_loop(0, _NCH, _dsc, 0)
            plsc.subcore_barrier()

            def _dga(b, cr):
                base = b * _K

                def _mkidx(j, c2):
                    sl = pl.ds(j * 16, 16)
                    didx_v[sl] = dst_v[pl.ds(base + j * 16, 16)]
                    return c2
                lax.fori_loop(0, _K // 16, _mkidx, 0)
                pltpu.sync_copy(den_sh.at[didx_v],
                                den_e.at[pl.ds(base, _K)])
                return cr
            lax.fori_loop(0, _NCH, _dga, 0)

            def _coef(i, cr):
                sl = pl.ds(i * 16, 16)
                ex_v[sl] = ex_v[sl] / (den_e[sl] + 1e-16)
                return cr
            lax.fori_loop(0, _EPP // 16, _coef, 0)

            goff = r * _NP

            def _st5(kk, cr):
                base = kk * _K

                def _mkidx(j, c2):
                    so = pl.ds(base + j * 16, 16)
                    sl = pl.ds(j * 16, 16)
                    idxg_v[sl] = src_v[so] + goff
                    didx_v[sl] = dst_v[so]
                    return c2
                lax.fori_loop(0, _K // 16, _mkidx, 0)

                @pl.when(c == 0)
                def _g0():
                    pltpu.async_copy(g2a_h.at[idxg_v], rows_v, sem).wait()

                @pl.when(c == 1)
                def _g1():
                    pltpu.async_copy(g2b_h.at[idxg_v], rows_v, sem).wait()

                def _scale(j, c2):
                    csp = plsc.load_gather(
                        ex_v, [jnp.zeros((16,), jnp.int32) + (base + j)])
                    for q in range(4):
                        sl = pl.ds(q * 16, 16)
                        rows_v[j, sl] = rows_v[j, sl] * csp
                    return c2
                lax.fori_loop(0, _K, _scale, 0)

                pltpu.sync_copy(rows_v, acc_sh.at[didx_v], add=True)
                return cr
            lax.fori_loop(0, _NCH, _st5, 0)
            plsc.subcore_barrier()

            srow = s * 640
            pltpu.sync_copy(acc_sh.at[pl.ds(srow, 640)],
                            h1t_h.at[pl.ds(c * _NP + srow, 640)])
            pltpu.sync_copy(z2_h, acc_sh.at[pl.ds(srow, 640)])
            plsc.subcore_barrier()

            hoff = c * _NP

            def _st6(kk, cr):
                base = kk * _K

                def _mkidx(j, c2):
                    so = pl.ds(base + j * 16, 16)
                    sl = pl.ds(j * 16, 16)
                    sidx_v[sl] = src_v[so] + hoff
                    didx_v[sl] = dst_v[so]
                    return c2
                lax.fori_loop(0, _K // 16, _mkidx, 0)

                pltpu.async_copy(h1t_h.at[sidx_v], rows_v, sem).wait()
                pltpu.sync_copy(rows_v, acc_sh.at[didx_v], add=True)
                return cr
            lax.fori_loop(0, _NCH, _st6, 0)
            plsc.subcore_barrier()

            row0 = s * 640
            pltpu.sync_copy(cnt_sh.at[pl.ds(row0, 640)], cnt_r)
            orow = (c * _R + r) * _NP + row0
            for q0, ln in ((0, 256), (256, 256), (512, 128)):
                pltpu.sync_copy(acc_sh.at[pl.ds(row0 + q0, ln)],
                                rows_v.at[pl.ds(0, ln)])

                def _fin(j, cr, q0=q0):
                    cs = plsc.load_gather(
                        cnt_r, [jnp.zeros((16,), jnp.int32) + (q0 + j)])
                    scale = 1.0 / jnp.maximum(cs, 1.0)
                    bsel = jnp.where(cs > 0.0, 1.0, 0.0)
                    for q in range(4):
                        sl = pl.ds(q * 16, 16)
                        rows_v[j, sl] = (rows_v[j, sl] * scale
                                         + bias_v[sl] * bsel)
                    return cr
                lax.fori_loop(0, ln, _fin, 0)
                pltpu.sync_copy(rows_v.at[pl.ds(0, ln)],
                                out_h.at[pl.ds(orow + q0, ln)])
            plsc.subcore_barrier()
            return carry
        lax.fori_loop(0, _R, _rel, 0)

    return k(srcp, dstp, as_flat, ad_flat, mh_flat, g2a, g2b, bias,
             zeros1, zeros2)


def kernel(tx_x, edge_index_card1, emb_card1, edge_index_card2, emb_card2,
           edge_index_card3, emb_card3, edge_index_card4, emb_card4,
           edge_index_card5, emb_card5, edge_index_card6, emb_card6,
           edge_index_ProductCD, emb_ProductCD,
           edge_index_P_emaildomain, emb_P_emaildomain,
           edge_index_addr1, emb_addr1, edge_index_addr2, emb_addr2,
           edge_index_dist1, emb_dist1,
           tx_W, tx_b, gat_W, gat_att_src, gat_att_dst, gat_bias,
           cls_W1, cls_b1, cls_W2, cls_b2, cls_W3, cls_b3):
    inp = dict(locals())
    edges = [inp['edge_index_' + t] for t in _ENTITY_TYPES]
    embs = [inp['emb_' + t] for t in _ENTITY_TYPES]

    src_all = jnp.stack([ei[0] for ei in edges])
    dst_all = jnp.stack([ei[1] for ei in edges])

    ENT = _ent_gather(src_all.reshape(-1), embs)

    G_all, ASD_all = _gasd(ENT, gat_W, gat_att_src, gat_att_dst)
    tx_h = _txh(tx_x, tx_W, tx_b)

    rowmask = (jnp.arange(_NP) < _N)[None, :]
    a_s = jnp.where(rowmask, ASD_all[:, 0].reshape(_R, _NP), 0.0)
    a_d = jnp.where(rowmask, ASD_all[:, 1].reshape(_R, _NP), 0.0)
    mh = (jnp.maximum(a_s.max(axis=1), 0.0)
          + jnp.maximum(a_d.max(axis=1), 0.0))
    mh_flat = jnp.broadcast_to(mh[:, None], (_R, 16)).reshape(-1)

    pad_src = jnp.arange(240, dtype=jnp.int32)
    pad_dst = (_N + jnp.arange(240)).astype(jnp.int32)
    src3 = src_all.reshape(_R, 16, _N)
    dst3 = dst_all.reshape(_R, 16, _N)
    srcp = jnp.concatenate(
        [src3, jnp.broadcast_to(pad_src, (_R, 16, 240))],
        axis=2).reshape(-1)
    dstp = jnp.concatenate(
        [dst3, jnp.broadcast_to(pad_dst, (_R, 16, 240))],
        axis=2).reshape(-1)

    zeros1 = jnp.zeros((640,), jnp.float32)
    zeros2 = jnp.zeros((640, 64), jnp.float32)

    out = _edge_sc(srcp, dstp, a_s.reshape(-1), a_d.reshape(-1), mh_flat,
                   G_all[:, :64], G_all[:, 64:], gat_bias, zeros1, zeros2)[0]

    out4 = out.reshape(2, _R, _NP, 64)
    msgs = [jnp.concatenate([out4[0, r, :_N], out4[1, r, :_N]], axis=1)
            for r in range(_R)]
    combined = jnp.concatenate([tx_h] + msgs, axis=1)

    return _mlp(combined, cls_W1, cls_b1, cls_W2, cls_b2, cls_W3, cls_b3)
